# Initial kernel scaffold; baseline (speedup 1.0000x reference)
import jax
import jax.numpy as jnp
from jax.experimental import pallas as pl


def kernel(x_nchw, w, b, gamma, beta):
    raise NotImplementedError("write your pallas kernel here")


# speedup vs baseline: 2.5414x; 2.5414x over previous
import functools

import jax
import jax.numpy as jnp
from jax.experimental import pallas as pl
from jax.experimental.pallas import tpu as pltpu

EPS = 1e-5
NEG_SLOPE = 0.2


def _conv_pool_stats_kernel(x_ref, wt_ref, b_ref, pooled_ref, stats_ref,
                            z_s, a_s):
    hw = x_ref.shape[2]
    h, wh, cout = a_s.shape

    z_s[...] = jax.lax.dot_general(
        x_ref[0], wt_ref[...], (((0,), (0,)), ((), ())),
        preferred_element_type=jnp.float32)

    wm = jnp.maximum(z_s[pl.dslice(0, hw // 2, 2)],
                     z_s[pl.dslice(1, hw // 2, 2)])
    a_s[...] = wm.reshape(h, wh, cout)

    pooled = jnp.maximum(a_s[pl.dslice(0, h // 2, 2)],
                         a_s[pl.dslice(1, h // 2, 2)])
    pooled = pooled.reshape((h // 2) * wh, cout) + b_ref[...]

    pooled_ref[0] = pooled
    stats_ref[0, 0:1, :] = jnp.sum(pooled, axis=0, keepdims=True)
    stats_ref[0, 1:2, :] = jnp.sum(pooled * pooled, axis=0, keepdims=True)


def _bn_lrelu_t_kernel(pooled_ref, stats_ref, g_ref, bt_ref, eye_ref,
                       out_ref, *, p_total):
    stats = stats_ref[...]
    s = jnp.sum(stats[:, 0, :], axis=0, keepdims=True)
    sq = jnp.sum(stats[:, 1, :], axis=0, keepdims=True)
    mean = s / p_total
    var = jnp.maximum(sq / p_total - mean * mean, 0.0)
    scale = g_ref[...] * jax.lax.rsqrt(var + EPS)
    shift = bt_ref[...] - mean * scale

    y = pooled_ref[0] * scale + shift
    y = jnp.where(y >= 0, y, NEG_SLOPE * y)

    out_ref[0] = jax.lax.dot_general(
        eye_ref[...], y, (((1,), (1,)), ((), ())),
        preferred_element_type=jnp.float32)


def kernel(x_nchw, w, b, gamma, beta):
    N, Cin, H, W = x_nchw.shape
    Cout = w.shape[0]
    Hh, Wh = H // 2, W // 2
    HW = H * W
    Q = Hh * Wh
    P = N * Q

    x3 = x_nchw.reshape(N, Cin, HW)
    wt = jnp.transpose(w, (1, 0))
    b2 = b.reshape(1, Cout)
    g2 = gamma.reshape(1, Cout)
    bt2 = beta.reshape(1, Cout)
    eye = jnp.eye(Cout, dtype=jnp.float32)

    par = pltpu.CompilerParams(dimension_semantics=("parallel",))

    pooled, stats = pl.pallas_call(
        _conv_pool_stats_kernel,
        grid=(N,),
        in_specs=[
            pl.BlockSpec((1, Cin, HW), lambda n: (n, 0, 0)),
            pl.BlockSpec((Cin, Cout), lambda n: (0, 0)),
            pl.BlockSpec((1, Cout), lambda n: (0, 0)),
        ],
        out_specs=(
            pl.BlockSpec((1, Q, Cout), lambda n: (n, 0, 0)),
            pl.BlockSpec((1, 2, Cout), lambda n: (n, 0, 0)),
        ),
        out_shape=(
            jax.ShapeDtypeStruct((N, Q, Cout), jnp.float32),
            jax.ShapeDtypeStruct((N, 2, Cout), jnp.float32),
        ),
        scratch_shapes=[
            pltpu.VMEM((HW, Cout), jnp.float32),
            pltpu.VMEM((H, Wh, Cout), jnp.float32),
        ],
        compiler_params=par,
    )(x3, wt, b2)

    out3 = pl.pallas_call(
        functools.partial(_bn_lrelu_t_kernel, p_total=float(P)),
        grid=(N,),
        in_specs=[
            pl.BlockSpec((1, Q, Cout), lambda n: (n, 0, 0)),
            pl.BlockSpec((N, 2, Cout), lambda n: (0, 0, 0)),
            pl.BlockSpec((1, Cout), lambda n: (0, 0)),
            pl.BlockSpec((1, Cout), lambda n: (0, 0)),
            pl.BlockSpec((Cout, Cout), lambda n: (0, 0)),
        ],
        out_specs=pl.BlockSpec((1, Cout, Q), lambda n: (n, 0, 0)),
        out_shape=jax.ShapeDtypeStruct((N, Cout, Q), jnp.float32),
        compiler_params=par,
    )(pooled, stats, g2, bt2, eye)

    return out3.reshape(N, Cout, Hh, Wh)
